# Initial kernel scaffold; baseline (speedup 1.0000x reference)
import jax
import jax.numpy as jnp
from jax.experimental import pallas as pl


def kernel(k_state, v_state, layer_idx, slice_indices, k_cache, v_cache):
    raise NotImplementedError("write your pallas kernel here")


# speedup vs baseline: 7.9574x; 7.9574x over previous
import functools

import jax
import jax.numpy as jnp
from jax import lax
from jax.experimental import pallas as pl
from jax.experimental.pallas import tpu as pltpu
from jax.experimental.pallas import tpu_sc as plsc

_L = 16
_CTX = 2048
_LAYERS = 2
_BATCH = 8


def _sc_kv_fetch(params_hbm, ks_hbm, vs_hbm, kc_hbm, vc_hbm,
                 ko_hbm, vo_hbm, pvm, slab):
    cid = lax.axis_index("c")
    sid = lax.axis_index("s")
    wid = sid * 2 + cid

    @pl.when(wid < 2 * _BATCH)
    def _work():
        pltpu.sync_copy(params_hbm, pvm)
        li_raw = pvm[pl.ds(0, _L)][0]
        begin_raw = pvm[pl.ds(_L, _L)][0]
        end_raw = pvm[pl.ds(2 * _L, _L)][0]
        li = jnp.clip(li_raw, 0, _LAYERS - 1)
        begin = jnp.clip(begin_raw, 0, _CTX - 1)
        p = jnp.clip(end_raw - 1, 0, _CTX - 1)
        hit = p == begin

        b = jnp.where(wid < _BATCH, wid, wid - _BATCH)
        is_k = wid < _BATCH

        @pl.when(is_k & hit)
        def _():
            pltpu.sync_copy(ks_hbm.at[b, 0], slab)
            pltpu.sync_copy(slab, ko_hbm.at[b, 0])

        @pl.when(is_k & jnp.logical_not(hit))
        def _():
            pltpu.sync_copy(kc_hbm.at[li, b, p], slab)
            pltpu.sync_copy(slab, ko_hbm.at[b, 0])

        @pl.when(jnp.logical_not(is_k) & hit)
        def _():
            pltpu.sync_copy(vs_hbm.at[b, 0], slab)
            pltpu.sync_copy(slab, vo_hbm.at[b, 0])

        @pl.when(jnp.logical_not(is_k) & jnp.logical_not(hit))
        def _():
            pltpu.sync_copy(vc_hbm.at[li, b, p], slab)
            pltpu.sync_copy(slab, vo_hbm.at[b, 0])


def kernel(k_state, v_state, layer_idx, slice_indices, k_cache, v_cache):
    si = slice_indices.astype(jnp.int32)
    li = jnp.asarray(layer_idx, jnp.int32)
    params = jnp.repeat(jnp.stack([li, si[0], si[1]]), _L)

    out_sds = jax.ShapeDtypeStruct(k_state.shape, k_state.dtype)
    mesh = plsc.VectorSubcoreMesh(core_axis_name="c", subcore_axis_name="s")
    run = pl.kernel(
        _sc_kv_fetch,
        mesh=mesh,
        out_type=(out_sds, out_sds),
        scratch_types=[
            pltpu.VMEM((3 * _L,), jnp.int32),
            pltpu.VMEM((_BATCH, 128), jnp.float32),
        ],
    )
    k_out, v_out = run(params, k_state, v_state, k_cache, v_cache)
    return (k_out, v_out)
